# baseline (speedup 1.0000x reference)
import jax
import jax.numpy as jnp
from jax.experimental import pallas as pl


def _add_body(m_ref, md_ref, r_ref, rd_ref, mo_ref, ro_ref):
    mo_ref[...] = m_ref[...] + md_ref[...]
    ro_ref[...] = r_ref[...] + rd_ref[...]


def kernel(means, scales, rot, means_def, rot_def):
    n = means.shape[0]
    R = 2000
    bs3 = pl.BlockSpec((R, 3), lambda i: (i, 0))
    bs4 = pl.BlockSpec((R, 4), lambda i: (i, 0))
    mo, ro = pl.pallas_call(
        _add_body,
        grid=(n // R,),
        in_specs=[bs3, bs3, bs4, bs4],
        out_specs=[bs3, bs4],
        out_shape=[
            jax.ShapeDtypeStruct((n, 3), means.dtype),
            jax.ShapeDtypeStruct((n, 4), rot.dtype),
        ],
    )(means, means_def, rot, rot_def)
    return (mo, scales, ro)
